# baseline (speedup 1.0000x reference)
import functools

import jax
import jax.numpy as jnp
from jax import lax
from jax.experimental import pallas as pl
from jax.experimental.pallas import tpu as pltpu
from jax.experimental.pallas import tpu_sc as plsc

TEXT_DIM = 768
HIDDEN = 256
B, L = 1024, 50
NC, NS = 2, 16
NW = NC * NS
CHUNK = 40

SLABS = 4
B_S = B // SLABS
TOK_S = B_S * L
NCHUNK = TOK_S // (NW * CHUNK)

BATCH_TILE = 64
TILE_T = BATCH_TILE * L
EPS = 1e-12


def _sc_gather_body(ids_hbm, table_hbm, out_hbm, idx_v,
                    rows0, rows1, gsem0, gsem1, wsem0, wsem1):
    wid = lax.axis_index("s") * NC + lax.axis_index("c")
    pltpu.sync_copy(ids_hbm.at[wid], idx_v)
    rows = (rows0, rows1)
    gsem = (gsem0, gsem1)
    wsem = (wsem0, wsem1)

    def dst(j):
        return out_hbm.at[pl.ds((wid * NCHUNK + j) * CHUNK, CHUNK)]

    gh = {0: pltpu.async_copy(table_hbm.at[idx_v.at[0]], rows[0], gsem[0])}
    wh = {}
    for j in range(NCHUNK):
        cur = j % 2
        nxt = 1 - cur
        gh[j].wait()
        if j + 1 < NCHUNK:
            if j >= 1:
                wh[j - 1].wait()
            gh[j + 1] = pltpu.async_copy(
                table_hbm.at[idx_v.at[j + 1]], rows[nxt], gsem[nxt])
        wh[j] = pltpu.async_copy(rows[cur], dst(j), wsem[cur])
    wh[NCHUNK - 2].wait()
    wh[NCHUNK - 1].wait()


@functools.cache
def _sc_gather():
    return pl.kernel(
        _sc_gather_body,
        mesh=plsc.VectorSubcoreMesh(core_axis_name="c", subcore_axis_name="s"),
        out_type=jax.ShapeDtypeStruct((TOK_S, TEXT_DIM), jnp.float32),
        scratch_types=[
            pltpu.VMEM((NCHUNK, CHUNK), jnp.int32),
            pltpu.VMEM((CHUNK, TEXT_DIM), jnp.float32),
            pltpu.VMEM((CHUNK, TEXT_DIM), jnp.float32),
            pltpu.SemaphoreType.DMA,
            pltpu.SemaphoreType.DMA,
            pltpu.SemaphoreType.DMA,
            pltpu.SemaphoreType.DMA,
        ],
    )


def _tc_body_first(x_ref, t_ref, w_ref, wt_ref, g_ref, b_ref, o_ref):
    _tc_compute(x_ref, t_ref, w_ref, wt_ref, g_ref, b_ref, o_ref)


def _tc_body_acc(prev_ref, x_ref, t_ref, w_ref, wt_ref, g_ref, b_ref, o_ref):
    del prev_ref
    _tc_compute(x_ref, t_ref, w_ref, wt_ref, g_ref, b_ref, o_ref)


def _tc_compute(x_ref, t_ref, w_ref, wt_ref, g_ref, b_ref, o_ref):
    x = x_ref[...].astype(jnp.bfloat16)
    t = t_ref[...]
    y = jnp.dot(x, w_ref[...].astype(jnp.bfloat16),
                preferred_element_type=jnp.float32)
    y = y + jnp.dot(t, wt_ref[...].astype(jnp.bfloat16),
                    preferred_element_type=jnp.float32)
    mu = jnp.mean(y, axis=-1, keepdims=True)
    d = y - mu
    var = jnp.mean(d * d, axis=-1, keepdims=True)
    z = d * lax.rsqrt(var + EPS) * g_ref[...] + b_ref[...]
    for b in range(BATCH_TILE):
        o_ref[b] = z[b * L:(b + 1) * L, :]


def _tc_specs(s):
    return dict(
        grid=(B_S // BATCH_TILE,),
        in_specs=[
            pl.BlockSpec((TILE_T, TEXT_DIM), lambda i: (i, 0)),
            pl.BlockSpec((TILE_T, 16),
                         lambda i, s=s: (s * (TOK_S // TILE_T) + i, 0)),
            pl.BlockSpec((TEXT_DIM, HIDDEN), lambda i: (0, 0)),
            pl.BlockSpec((16, HIDDEN), lambda i: (0, 0)),
            pl.BlockSpec((1, HIDDEN), lambda i: (0, 0)),
            pl.BlockSpec((1, HIDDEN), lambda i: (0, 0)),
        ],
        out_specs=pl.BlockSpec(
            (BATCH_TILE, L, HIDDEN),
            lambda i, s=s: (s * (B_S // BATCH_TILE) + i, 0, 0)),
        out_shape=jax.ShapeDtypeStruct((B, L, HIDDEN), jnp.float32),
    )


@functools.cache
def _tc_call(s):
    spec = _tc_specs(s)
    if s == 0:
        return pl.pallas_call(_tc_body_first, **spec)
    spec["in_specs"] = [pl.BlockSpec(memory_space=pl.ANY)] + spec["in_specs"]
    return pl.pallas_call(_tc_body_acc, input_output_aliases={0: 0}, **spec)


def kernel(input_ids, rel_ids, token_type_ids, token_table, W_proj, b_proj,
           rel_table, type_table, ln_gamma, ln_beta):
    ids4 = input_ids.astype(jnp.int32).reshape(SLABS, NW, NCHUNK, CHUNK)

    lanes = jnp.arange(16, dtype=jnp.int32)
    tails = ((lanes[None, :] == rel_ids.reshape(-1, 1))
             | (lanes[None, :] == token_type_ids.reshape(-1, 1) + 3)
             | (lanes[None, :] == 5)).astype(jnp.bfloat16)
    w_tail = jnp.concatenate(
        [rel_table, type_table, b_proj[None, :],
         jnp.zeros((16 - 3 - 2 - 1, HIDDEN), jnp.float32)], axis=0)
    gamma, beta = ln_gamma[None, :], ln_beta[None, :]

    out = None
    for s in range(SLABS):
        gathered = _sc_gather()(ids4[s], token_table)
        args = (gathered, tails, W_proj, w_tail, gamma, beta)
        out = _tc_call(s)(*args) if s == 0 else _tc_call(s)(out, *args)
    return out


# speedup vs baseline: 1.0214x; 1.0214x over previous
import functools

import jax
import jax.numpy as jnp
from jax import lax
from jax.experimental import pallas as pl
from jax.experimental.pallas import tpu as pltpu
from jax.experimental.pallas import tpu_sc as plsc

TEXT_DIM = 768
HIDDEN = 256
B, L = 1024, 50
NC, NS = 2, 16
NW = NC * NS
CHUNK = 80

SLABS = 4
B_S = B // SLABS
TOK_S = B_S * L
NCHUNK = TOK_S // (NW * CHUNK)

BATCH_TILE = 64
TILE_T = BATCH_TILE * L
EPS = 1e-12


def _sc_gather_body(ids_hbm, table_hbm, out_hbm, idx_v,
                    rows0, rows1, gsem0, gsem1, wsem0, wsem1):
    wid = lax.axis_index("s") * NC + lax.axis_index("c")
    pltpu.sync_copy(ids_hbm.at[wid], idx_v)
    rows = (rows0, rows1)
    gsem = (gsem0, gsem1)
    wsem = (wsem0, wsem1)

    def dst(j):
        return out_hbm.at[pl.ds((wid * NCHUNK + j) * CHUNK, CHUNK)]

    gh = {0: pltpu.async_copy(table_hbm.at[idx_v.at[0]], rows[0], gsem[0])}
    wh = {}
    for j in range(NCHUNK):
        cur = j % 2
        nxt = 1 - cur
        gh[j].wait()
        if j + 1 < NCHUNK:
            if j >= 1:
                wh[j - 1].wait()
            gh[j + 1] = pltpu.async_copy(
                table_hbm.at[idx_v.at[j + 1]], rows[nxt], gsem[nxt])
        wh[j] = pltpu.async_copy(rows[cur], dst(j), wsem[cur])
    wh[NCHUNK - 2].wait()
    wh[NCHUNK - 1].wait()


@functools.cache
def _sc_gather():
    return pl.kernel(
        _sc_gather_body,
        mesh=plsc.VectorSubcoreMesh(core_axis_name="c", subcore_axis_name="s"),
        out_type=jax.ShapeDtypeStruct((TOK_S, TEXT_DIM), jnp.float32),
        scratch_types=[
            pltpu.VMEM((NCHUNK, CHUNK), jnp.int32),
            pltpu.VMEM((CHUNK, TEXT_DIM), jnp.float32),
            pltpu.VMEM((CHUNK, TEXT_DIM), jnp.float32),
            pltpu.SemaphoreType.DMA,
            pltpu.SemaphoreType.DMA,
            pltpu.SemaphoreType.DMA,
            pltpu.SemaphoreType.DMA,
        ],
    )


def _tc_body_first(x_ref, t_ref, w_ref, wt_ref, g_ref, b_ref, o_ref):
    _tc_compute(x_ref, t_ref, w_ref, wt_ref, g_ref, b_ref, o_ref)


def _tc_body_acc(prev_ref, x_ref, t_ref, w_ref, wt_ref, g_ref, b_ref, o_ref):
    del prev_ref
    _tc_compute(x_ref, t_ref, w_ref, wt_ref, g_ref, b_ref, o_ref)


def _tc_compute(x_ref, t_ref, w_ref, wt_ref, g_ref, b_ref, o_ref):
    x = x_ref[...].astype(jnp.bfloat16)
    t = t_ref[...]
    y = jnp.dot(x, w_ref[...].astype(jnp.bfloat16),
                preferred_element_type=jnp.float32)
    y = y + jnp.dot(t, wt_ref[...].astype(jnp.bfloat16),
                    preferred_element_type=jnp.float32)
    mu = jnp.mean(y, axis=-1, keepdims=True)
    d = y - mu
    var = jnp.mean(d * d, axis=-1, keepdims=True)
    z = d * lax.rsqrt(var + EPS) * g_ref[...] + b_ref[...]
    for b in range(BATCH_TILE):
        o_ref[b] = z[b * L:(b + 1) * L, :]


def _tc_specs(s):
    return dict(
        grid=(B_S // BATCH_TILE,),
        in_specs=[
            pl.BlockSpec((TILE_T, TEXT_DIM), lambda i: (i, 0)),
            pl.BlockSpec((TILE_T, 16),
                         lambda i, s=s: (s * (TOK_S // TILE_T) + i, 0)),
            pl.BlockSpec((TEXT_DIM, HIDDEN), lambda i: (0, 0)),
            pl.BlockSpec((16, HIDDEN), lambda i: (0, 0)),
            pl.BlockSpec((1, HIDDEN), lambda i: (0, 0)),
            pl.BlockSpec((1, HIDDEN), lambda i: (0, 0)),
        ],
        out_specs=pl.BlockSpec(
            (BATCH_TILE, L, HIDDEN),
            lambda i, s=s: (s * (B_S // BATCH_TILE) + i, 0, 0)),
        out_shape=jax.ShapeDtypeStruct((B, L, HIDDEN), jnp.float32),
    )


@functools.cache
def _tc_call(s):
    spec = _tc_specs(s)
    if s == 0:
        return pl.pallas_call(_tc_body_first, **spec)
    spec["in_specs"] = [pl.BlockSpec(memory_space=pl.ANY)] + spec["in_specs"]
    return pl.pallas_call(_tc_body_acc, input_output_aliases={0: 0}, **spec)


def kernel(input_ids, rel_ids, token_type_ids, token_table, W_proj, b_proj,
           rel_table, type_table, ln_gamma, ln_beta):
    ids4 = input_ids.astype(jnp.int32).reshape(SLABS, NW, NCHUNK, CHUNK)

    lanes = jnp.arange(16, dtype=jnp.int32)
    tails = ((lanes[None, :] == rel_ids.reshape(-1, 1))
             | (lanes[None, :] == token_type_ids.reshape(-1, 1) + 3)
             | (lanes[None, :] == 5)).astype(jnp.bfloat16)
    w_tail = jnp.concatenate(
        [rel_table, type_table, b_proj[None, :],
         jnp.zeros((16 - 3 - 2 - 1, HIDDEN), jnp.float32)], axis=0)
    gamma, beta = ln_gamma[None, :], ln_beta[None, :]

    out = None
    for s in range(SLABS):
        gathered = _sc_gather()(ids4[s], token_table)
        args = (gathered, tails, W_proj, w_tail, gamma, beta)
        out = _tc_call(s)(*args) if s == 0 else _tc_call(s)(out, *args)
    return out


# speedup vs baseline: 1.0371x; 1.0154x over previous
import functools

import jax
import jax.numpy as jnp
from jax import lax
from jax.experimental import pallas as pl
from jax.experimental.pallas import tpu as pltpu
from jax.experimental.pallas import tpu_sc as plsc

TEXT_DIM = 768
HIDDEN = 256
B, L = 1024, 50
NC, NS = 2, 16
NW = NC * NS
CHUNK = 80

SLABS = 4
B_S = B // SLABS
TOK_S = B_S * L
NCHUNK = TOK_S // (NW * CHUNK)

BATCH_TILE = 64
TILE_T = BATCH_TILE * L
EPS = 1e-12


def _sc_gather_body(ids_hbm, table_hbm, out_hbm, idx_v,
                    rows0, rows1, gsem0, gsem1, wsem0, wsem1):
    wid = lax.axis_index("s") * NC + lax.axis_index("c")
    pltpu.sync_copy(ids_hbm.at[wid], idx_v)
    rows = (rows0, rows1)
    gsem = (gsem0, gsem1)
    wsem = (wsem0, wsem1)

    def dst(j):
        return out_hbm.at[pl.ds((wid * NCHUNK + j) * CHUNK, CHUNK)]

    gh = {0: pltpu.async_copy(table_hbm.at[idx_v.at[0]], rows[0], gsem[0])}
    wh = {}
    for j in range(NCHUNK):
        cur = j % 2
        nxt = 1 - cur
        gh[j].wait()
        if j + 1 < NCHUNK:
            if j >= 1:
                wh[j - 1].wait()
            gh[j + 1] = pltpu.async_copy(
                table_hbm.at[idx_v.at[j + 1]], rows[nxt], gsem[nxt])
        wh[j] = pltpu.async_copy(rows[cur], dst(j), wsem[cur])
    wh[NCHUNK - 2].wait()
    wh[NCHUNK - 1].wait()


@functools.cache
def _sc_gather():
    return pl.kernel(
        _sc_gather_body,
        mesh=plsc.VectorSubcoreMesh(core_axis_name="c", subcore_axis_name="s"),
        out_type=jax.ShapeDtypeStruct((TOK_S, TEXT_DIM), jnp.float32),
        scratch_types=[
            pltpu.VMEM((NCHUNK, CHUNK), jnp.int32),
            pltpu.VMEM((CHUNK, TEXT_DIM), jnp.float32),
            pltpu.VMEM((CHUNK, TEXT_DIM), jnp.float32),
            pltpu.SemaphoreType.DMA,
            pltpu.SemaphoreType.DMA,
            pltpu.SemaphoreType.DMA,
            pltpu.SemaphoreType.DMA,
        ],
    )


def _tc_body_first(x_ref, t_ref, w_ref, wt_ref, g_ref, b_ref, o_ref):
    _tc_compute(x_ref, t_ref, w_ref, wt_ref, g_ref, b_ref, o_ref)


def _tc_body_acc(prev_ref, x_ref, t_ref, w_ref, wt_ref, g_ref, b_ref, o_ref):
    del prev_ref
    _tc_compute(x_ref, t_ref, w_ref, wt_ref, g_ref, b_ref, o_ref)


def _tc_compute(x_ref, t_ref, w_ref, wt_ref, g_ref, b_ref, o_ref):
    x = x_ref[...].astype(jnp.bfloat16)
    t = t_ref[...].astype(jnp.bfloat16)
    y = jnp.dot(x, w_ref[...].astype(jnp.bfloat16),
                preferred_element_type=jnp.float32)
    y = y + jnp.dot(t, wt_ref[...].astype(jnp.bfloat16),
                    preferred_element_type=jnp.float32)
    mu = jnp.mean(y, axis=-1, keepdims=True)
    d = y - mu
    var = jnp.mean(d * d, axis=-1, keepdims=True)
    z = d * lax.rsqrt(var + EPS) * g_ref[...] + b_ref[...]
    for b in range(BATCH_TILE):
        o_ref[b] = z[b * L:(b + 1) * L, :]


def _tc_specs(s):
    return dict(
        grid=(B_S // BATCH_TILE,),
        in_specs=[
            pl.BlockSpec((TILE_T, TEXT_DIM), lambda i: (i, 0)),
            pl.BlockSpec((TILE_T, 16),
                         lambda i, s=s: (s * (TOK_S // TILE_T) + i, 0)),
            pl.BlockSpec((TEXT_DIM, HIDDEN), lambda i: (0, 0)),
            pl.BlockSpec((16, HIDDEN), lambda i: (0, 0)),
            pl.BlockSpec((1, HIDDEN), lambda i: (0, 0)),
            pl.BlockSpec((1, HIDDEN), lambda i: (0, 0)),
        ],
        out_specs=pl.BlockSpec(
            (BATCH_TILE, L, HIDDEN),
            lambda i, s=s: (s * (B_S // BATCH_TILE) + i, 0, 0)),
        out_shape=jax.ShapeDtypeStruct((B, L, HIDDEN), jnp.float32),
    )


@functools.cache
def _tc_call(s):
    spec = _tc_specs(s)
    if s == 0:
        return pl.pallas_call(_tc_body_first, **spec)
    spec["in_specs"] = [pl.BlockSpec(memory_space=pl.ANY)] + spec["in_specs"]
    return pl.pallas_call(_tc_body_acc, input_output_aliases={0: 0}, **spec)


def kernel(input_ids, rel_ids, token_type_ids, token_table, W_proj, b_proj,
           rel_table, type_table, ln_gamma, ln_beta):
    ids4 = input_ids.astype(jnp.int32).reshape(SLABS, NW, NCHUNK, CHUNK)

    lanes = jnp.arange(16, dtype=jnp.int32)
    tails = ((lanes[None, :] == rel_ids.reshape(-1, 1))
             | (lanes[None, :] == token_type_ids.reshape(-1, 1) + 3)
             | (lanes[None, :] == 5)).astype(jnp.int8)
    w_tail = jnp.concatenate(
        [rel_table, type_table, b_proj[None, :],
         jnp.zeros((16 - 3 - 2 - 1, HIDDEN), jnp.float32)], axis=0)
    gamma, beta = ln_gamma[None, :], ln_beta[None, :]

    out = None
    for s in range(SLABS):
        gathered = _sc_gather()(ids4[s], token_table)
        args = (gathered, tails, W_proj, w_tail, gamma, beta)
        out = _tc_call(s)(*args) if s == 0 else _tc_call(s)(out, *args)
    return out
